# baseline (speedup 1.0000x reference)
import functools

import jax
import jax.numpy as jnp
from jax import lax
from jax.experimental import pallas as pl
from jax.experimental.pallas import tpu as pltpu
from jax.experimental.pallas import tpu_sc as plsc

_N = 10000
_E = 320000
_D = 128
_H = 512
_K = 20

_NC = 2
_NS = 16
_L = 16

_EPS = _E // _NS
_CH = 80
_GC = 25
_GE = _GC * _CH
_NG = _EPS // _GE
_SPS = 624
_WBC = ((0, 80), (80, 80), (160, 80), (240, 80), (320, 80), (400, 80),
        (480, 80), (560, 64))
_TAIL = _N - _NS * _SPS

@functools.cache
def _get_spmm():
    mesh = plsc.VectorSubcoreMesh(core_axis_name="c", subcore_axis_name="s",
                                  num_cores=_NC, num_subcores=_NS)
    return pl.kernel(
        _spmm_body,
        out_type=jax.ShapeDtypeStruct((_NC, _N, _D), jnp.float32),
        mesh=mesh,
        compiler_params=pltpu.CompilerParams(needs_layout_passes=False,
                                             use_tc_tiling_on_sc=False),
        scratch_types=[
            pltpu.VMEM_SHARED((_N, _D), jnp.float32),
            pltpu.VMEM((_GC, _CH), jnp.int32),
            pltpu.VMEM((_GC, _CH), jnp.int32),
            pltpu.VMEM((_GE,), jnp.float32),
            pltpu.VMEM((_CH, _D // 2), jnp.int32),
            pltpu.VMEM((_CH, _D // 2), jnp.int32),
            pltpu.VMEM((_CH, _D // 2), jnp.int32),
            pltpu.VMEM((_CH, _D), jnp.float32),
            pltpu.VMEM((_CH, _D), jnp.float32),
            pltpu.SemaphoreType.DMA,
            pltpu.SemaphoreType.DMA,
            pltpu.SemaphoreType.DMA,
            pltpu.SemaphoreType.DMA,
            pltpu.SemaphoreType.DMA,
        ],
    )


def _spmm_body(seq1h, seq2h, srch, dsth, wh, outh, acc, srcv, dstv, wv,
               gb0, gb1, gb2, st0, st1, gsem0, gsem1, gsem2, ssem0, ssem1):
    s = lax.axis_index("s")
    c = lax.axis_index("c")

    def zrow(j, carry):
        for k2 in range(_D // _L):
            st0[j, k2 * _L:(k2 + 1) * _L] = jnp.zeros((_L,), jnp.float32)
        return carry

    lax.fori_loop(0, _CH, zrow, 0)
    base = s * _SPS
    for off, nr in _WBC:
        pltpu.sync_copy(st0.at[pl.ds(0, nr)],
                        acc.at[pl.ds(base + off, nr)])

    @pl.when(s == 0)
    def _():
        pltpu.sync_copy(st0.at[pl.ds(0, _TAIL)],
                        acc.at[pl.ds(_NS * _SPS, _TAIL)])

    plsc.subcore_barrier()

    def core_prog(seqh, out2):
        gbufs = ((gb0, gsem0), (gb1, gsem1), (gb2, gsem2))
        sbufs = ((st0, ssem0), (st1, ssem1))

        def group(g, carry):
            pltpu.sync_copy(srch.at[s, g], srcv)
            pltpu.sync_copy(dsth.at[s, g], dstv)
            pltpu.sync_copy(wh.at[s, g], wv)
            pltpu.async_copy(seqh.at[srcv.at[0]], gb0, gsem0)
            pltpu.async_copy(seqh.at[srcv.at[1]], gb1, gsem1)
            pltpu.async_copy(seqh.at[srcv.at[2]], gb2, gsem2)

            def chunk(t, c3):
                for b6 in range(6):
                    @pl.when(t % 6 == b6)
                    def _():
                        gb, gs = gbufs[b6 % 3]
                        sb, ss = sbufs[b6 % 2]
                        pltpu.make_async_copy(
                            seqh.at[srcv.at[t]], gb, gs).wait()

                        @pl.when(t >= 2)
                        def _():
                            pltpu.make_async_copy(
                                sb, acc.at[dstv.at[0]], ss).wait()

                        @plsc.parallel_loop(0, _CH, unroll=4)
                        def _(i):
                            wspl = plsc.load_gather(
                                wv, [jnp.broadcast_to(t * _CH + i, (_L,))])
                            for k2 in range(_D // 32):
                                x = gb[i, _L * k2:_L * (k2 + 1)]
                                fe = plsc.bitcast(x << 16, jnp.float32)
                                fo = plsc.bitcast(
                                    x & jnp.int32(-65536), jnp.float32)
                                sb[i, 32 * k2:32 * k2 + _L] = fe * wspl
                                sb[i, 32 * k2 + _L:32 * (k2 + 1)] = fo * wspl

                        pltpu.async_copy(sb, acc.at[dstv.at[t]], ss, add=True)

                        @pl.when(t + 3 < _GC)
                        def _():
                            pltpu.async_copy(
                                seqh.at[srcv.at[t + 3]], gb, gs)
                return c3

            lax.fori_loop(0, _GC, chunk, 0)
            for u in range(_GC - 2, _GC):
                sb, ss = sbufs[u % 2]
                pltpu.make_async_copy(sb, acc.at[dstv.at[0]], ss).wait()
            return carry

        lax.fori_loop(0, _NG, group, 0)
        plsc.subcore_barrier()
        for off, nr in _WBC:
            r0 = s * _SPS + off
            pltpu.sync_copy(acc.at[pl.ds(r0, nr)], st0.at[pl.ds(0, nr)])
            pltpu.sync_copy(st0.at[pl.ds(0, nr)], out2.at[pl.ds(r0, nr)])

        @pl.when(s == 0)
        def _():
            pltpu.sync_copy(acc.at[pl.ds(_NS * _SPS, _TAIL)],
                            st0.at[pl.ds(0, _TAIL)])
            pltpu.sync_copy(st0.at[pl.ds(0, _TAIL)],
                            out2.at[pl.ds(_NS * _SPS, _TAIL)])

    @pl.when(c == 0)
    def _():
        core_prog(seq1h, outh.at[0])

    @pl.when(c == 1)
    def _():
        core_prog(seq2h, outh.at[1])


_R = 1000
_G = _N // _R


def _dense1_body(agg1_ref, agg2_ref, w_ref, b_ref, alpha_ref, h1_ref, h2_ref,
                 hsum_ref):
    a = alpha_ref[0]
    w = w_ref[...]
    b = b_ref[...]
    y1 = jnp.dot(agg1_ref[0], w, preferred_element_type=jnp.float32) + b
    h1 = jnp.where(y1 >= 0, y1, a * y1)
    h1_ref[...] = h1
    y2 = jnp.dot(agg2_ref[0], w, preferred_element_type=jnp.float32) + b
    h2_ref[...] = jnp.where(y2 >= 0, y2, a * y2)

    @pl.when(pl.program_id(0) == 0)
    def _():
        hsum_ref[...] = jnp.zeros_like(hsum_ref)

    hsum_ref[...] += jnp.sum(h1, axis=0, keepdims=True)


_dense1 = pl.pallas_call(
    _dense1_body,
    grid=(_G,),
    in_specs=[
        pl.BlockSpec((1, _R, _D), lambda i: (0, i, 0)),
        pl.BlockSpec((1, _R, _D), lambda i: (1, i, 0)),
        pl.BlockSpec((_D, _H), lambda i: (0, 0)),
        pl.BlockSpec((1, _H), lambda i: (0, 0)),
        pl.BlockSpec(memory_space=pltpu.SMEM),
    ],
    out_specs=[
        pl.BlockSpec((_R, _H), lambda i: (i, 0)),
        pl.BlockSpec((_R, _H), lambda i: (i, 0)),
        pl.BlockSpec((1, _H), lambda i: (0, 0)),
    ],
    out_shape=[
        jax.ShapeDtypeStruct((_N, _H), jnp.float32),
        jax.ShapeDtypeStruct((_N, _H), jnp.float32),
        jax.ShapeDtypeStruct((1, _H), jnp.float32),
    ],
)


def _dense2_body(h1_ref, h2_ref, hsum_ref, wd_ref, cl_ref, sb1_ref, sb2_ref,
                 bd_ref, sc1_ref, sc2_ref, q_ref):
    cvec = jax.nn.sigmoid(hsum_ref[...] / _N)
    v = lax.dot_general(wd_ref[...], cvec, (((1,), (1,)), ((), ())),
                        preferred_element_type=jnp.float32)
    h1 = h1_ref[...]
    h2 = h2_ref[...]
    bd = bd_ref[0]
    sc1_ref[...] = (jnp.dot(h1, v, preferred_element_type=jnp.float32)
                    + bd + sb1_ref[...])
    sc2_ref[...] = (jnp.dot(h2, v, preferred_element_type=jnp.float32)
                    + bd + sb2_ref[...])
    cl = cl_ref[...]
    cross = lax.dot_general(h1, cl, (((1,), (1,)), ((), ())),
                            preferred_element_type=jnp.float32)
    h1s = jnp.sum(h1 * h1, axis=1, keepdims=True)
    cls = lax.dot_general(jnp.ones((1, _H), jnp.float32), cl * cl,
                          (((1,), (1,)), ((), ())),
                          preferred_element_type=jnp.float32)
    dist2 = h1s - 2.0 * cross + cls
    qn = 1.0 / (1.0 + dist2)
    q_ref[...] = qn / jnp.sum(qn, axis=1, keepdims=True)


_dense2 = pl.pallas_call(
    _dense2_body,
    grid=(_G,),
    in_specs=[
        pl.BlockSpec((_R, _H), lambda i: (i, 0)),
        pl.BlockSpec((_R, _H), lambda i: (i, 0)),
        pl.BlockSpec((1, _H), lambda i: (0, 0)),
        pl.BlockSpec((_H, _H), lambda i: (0, 0)),
        pl.BlockSpec((_K, _H), lambda i: (0, 0)),
        pl.BlockSpec((_R, 1), lambda i: (i, 0)),
        pl.BlockSpec((_R, 1), lambda i: (i, 0)),
        pl.BlockSpec(memory_space=pltpu.SMEM),
    ],
    out_specs=[
        pl.BlockSpec((_R, 1), lambda i: (i, 0)),
        pl.BlockSpec((_R, 1), lambda i: (i, 0)),
        pl.BlockSpec((_R, _K), lambda i: (i, 0)),
    ],
    out_shape=[
        jax.ShapeDtypeStruct((_N, 1), jnp.float32),
        jax.ShapeDtypeStruct((_N, 1), jnp.float32),
        jax.ShapeDtypeStruct((_N, _K), jnp.float32),
    ],
)


def kernel(seq1, seq2, adj_edge_index, adj_edge_weight, samp_bias1, samp_bias2,
           W_gcn, b_gcn, alpha, W_disc, b_disc, cluster_layer):
    s1 = lax.bitcast_convert_type(
        seq1[0].astype(jnp.bfloat16).reshape(_N, _D // 2, 2), jnp.int32)
    s2 = lax.bitcast_convert_type(
        seq2[0].astype(jnp.bfloat16).reshape(_N, _D // 2, 2), jnp.int32)
    src = adj_edge_index[1].astype(jnp.int32).reshape(_NS, _NG, _GC, _CH)
    dst = adj_edge_index[0].astype(jnp.int32).reshape(_NS, _NG, _GC, _CH)
    w = adj_edge_weight.astype(jnp.float32).reshape(_NS, _NG, _GE)
    wp = W_gcn.reshape(_D // 32, 16, 2, _H).transpose(0, 2, 1, 3)
    wp = wp.reshape(_D, _H)

    agg = _get_spmm()(s1, s2, src, dst, w)
    h1, h2, hsum = _dense1(agg, agg, wp,
                           b_gcn.reshape(1, _H), alpha.reshape(1))
    sc1, sc2, q = _dense2(h1, h2, hsum, W_disc, cluster_layer,
                          samp_bias1.reshape(_N, 1), samp_bias2.reshape(_N, 1),
                          b_disc.reshape(1))
    ret = jnp.concatenate([sc1.reshape(1, _N), sc2.reshape(1, _N)], axis=1)
    return (ret, q, h1)


# speedup vs baseline: 1.1721x; 1.1721x over previous
import functools

import jax
import jax.numpy as jnp
from jax import lax
from jax.experimental import pallas as pl
from jax.experimental.pallas import tpu as pltpu
from jax.experimental.pallas import tpu_sc as plsc

_N = 10000
_E = 320000
_D = 128
_H = 512
_K = 20

_NC = 2
_NS = 16
_L = 16

_EPS = _E // _NS
_CH = 80
_GC = 25
_GE = _GC * _CH
_NG = _EPS // _GE
_SPS = 624
_WBC = ((0, 80), (80, 80), (160, 80), (240, 80), (320, 80), (400, 80),
        (480, 80), (560, 64))
_TAIL = _N - _NS * _SPS

@functools.cache
def _get_spmm():
    mesh = plsc.VectorSubcoreMesh(core_axis_name="c", subcore_axis_name="s",
                                  num_cores=_NC, num_subcores=_NS)
    return pl.kernel(
        _spmm_body,
        out_type=jax.ShapeDtypeStruct((_NC, _N, _D), jnp.float32),
        mesh=mesh,
        compiler_params=pltpu.CompilerParams(needs_layout_passes=False,
                                             use_tc_tiling_on_sc=False),
        scratch_types=[
            pltpu.VMEM_SHARED((_N, _D), jnp.float32),
            pltpu.VMEM((_GC, _CH), jnp.int32),
            pltpu.VMEM((_GC, _CH), jnp.int32),
            pltpu.VMEM((_GE,), jnp.float32),
            pltpu.VMEM((_CH, _D // 2), jnp.int32),
            pltpu.VMEM((_CH, _D // 2), jnp.int32),
            pltpu.VMEM((_CH, _D // 2), jnp.int32),
            pltpu.VMEM((_CH, _D), jnp.float32),
            pltpu.VMEM((_CH, _D), jnp.float32),
            pltpu.SemaphoreType.DMA,
            pltpu.SemaphoreType.DMA,
            pltpu.SemaphoreType.DMA,
            pltpu.SemaphoreType.DMA,
            pltpu.SemaphoreType.DMA,
        ],
    )


def _spmm_body(seq1h, seq2h, eih, wh, outh, acc, dstv, srcv, wv,
               gb0, gb1, gb2, st0, st1, gsem0, gsem1, gsem2, ssem0, ssem1):
    s = lax.axis_index("s")
    c = lax.axis_index("c")

    def zrow(j, carry):
        for k2 in range(_D // _L):
            st0[j, k2 * _L:(k2 + 1) * _L] = jnp.zeros((_L,), jnp.float32)
        return carry

    lax.fori_loop(0, _CH, zrow, 0)
    base = s * _SPS
    for off, nr in _WBC:
        pltpu.sync_copy(st0.at[pl.ds(0, nr)],
                        acc.at[pl.ds(base + off, nr)])

    @pl.when(s == 0)
    def _():
        pltpu.sync_copy(st0.at[pl.ds(0, _TAIL)],
                        acc.at[pl.ds(_NS * _SPS, _TAIL)])

    plsc.subcore_barrier()

    def core_prog(seqh, out2):
        gbufs = ((gb0, gsem0), (gb1, gsem1), (gb2, gsem2))
        sbufs = ((st0, ssem0), (st1, ssem1))

        def group(g, carry):
            pltpu.sync_copy(eih.at[0, s, g], dstv)
            pltpu.sync_copy(eih.at[1, s, g], srcv)
            pltpu.sync_copy(wh.at[s, g], wv)
            pltpu.async_copy(seqh.at[srcv.at[0]], gb0, gsem0)
            pltpu.async_copy(seqh.at[srcv.at[1]], gb1, gsem1)
            pltpu.async_copy(seqh.at[srcv.at[2]], gb2, gsem2)

            def chunk(t, c3):
                for b6 in range(6):
                    @pl.when(t % 6 == b6)
                    def _():
                        gb, gs = gbufs[b6 % 3]
                        sb, ss = sbufs[b6 % 2]
                        pltpu.make_async_copy(
                            seqh.at[srcv.at[t]], gb, gs).wait()

                        @pl.when(t >= 2)
                        def _():
                            pltpu.make_async_copy(
                                sb, acc.at[dstv.at[0]], ss).wait()

                        @plsc.parallel_loop(0, _CH, unroll=4)
                        def _(i):
                            wspl = plsc.load_gather(
                                wv, [jnp.broadcast_to(t * _CH + i, (_L,))])
                            for k2 in range(_D // 32):
                                x = gb[i, _L * k2:_L * (k2 + 1)]
                                fe = plsc.bitcast(x << 16, jnp.float32)
                                fo = plsc.bitcast(
                                    x & jnp.int32(-65536), jnp.float32)
                                sb[i, 32 * k2:32 * k2 + _L] = fe * wspl
                                sb[i, 32 * k2 + _L:32 * (k2 + 1)] = fo * wspl

                        pltpu.async_copy(sb, acc.at[dstv.at[t]], ss, add=True)

                        @pl.when(t + 3 < _GC)
                        def _():
                            pltpu.async_copy(
                                seqh.at[srcv.at[t + 3]], gb, gs)
                return c3

            lax.fori_loop(0, _GC, chunk, 0)
            for u in range(_GC - 2, _GC):
                sb, ss = sbufs[u % 2]
                pltpu.make_async_copy(sb, acc.at[dstv.at[0]], ss).wait()
            return carry

        lax.fori_loop(0, _NG, group, 0)
        plsc.subcore_barrier()
        for off, nr in _WBC:
            r0 = s * _SPS + off
            pltpu.sync_copy(acc.at[pl.ds(r0, nr)], st0.at[pl.ds(0, nr)])
            pltpu.sync_copy(st0.at[pl.ds(0, nr)], out2.at[pl.ds(r0, nr)])

        @pl.when(s == 0)
        def _():
            pltpu.sync_copy(acc.at[pl.ds(_NS * _SPS, _TAIL)],
                            st0.at[pl.ds(0, _TAIL)])
            pltpu.sync_copy(st0.at[pl.ds(0, _TAIL)],
                            out2.at[pl.ds(_NS * _SPS, _TAIL)])

    @pl.when(c == 0)
    def _():
        core_prog(seq1h, outh.at[0])

    @pl.when(c == 1)
    def _():
        core_prog(seq2h, outh.at[1])


_R = 1000
_G = _N // _R


def _rne_bf16_bits(v):
    u = lax.bitcast_convert_type(v, jnp.int32)
    r = u + jnp.int32(0x7FFF) + ((u >> 16) & jnp.int32(1))
    return r


def _prep_body(s1_ref, s2_ref, t1_ref, t2_ref):
    for x_ref, t_ref in ((s1_ref, t1_ref), (s2_ref, t2_ref)):
        x = x_ref[0]
        xl = jnp.concatenate(
            [x[:, 32 * k:32 * k + 16] for k in range(_D // 32)], axis=1)
        xh = jnp.concatenate(
            [x[:, 32 * k + 16:32 * k + 32] for k in range(_D // 32)], axis=1)
        lb = (_rne_bf16_bits(xl) >> 16) & jnp.int32(0xFFFF)
        hb = _rne_bf16_bits(xh) & jnp.int32(-65536)
        t_ref[...] = hb | lb


_prep = pl.pallas_call(
    _prep_body,
    grid=(_G,),
    in_specs=[
        pl.BlockSpec((1, _R, _D), lambda i: (0, i, 0)),
        pl.BlockSpec((1, _R, _D), lambda i: (0, i, 0)),
    ],
    out_specs=[
        pl.BlockSpec((_R, _D // 2), lambda i: (i, 0)),
        pl.BlockSpec((_R, _D // 2), lambda i: (i, 0)),
    ],
    out_shape=[
        jax.ShapeDtypeStruct((_N, _D // 2), jnp.int32),
        jax.ShapeDtypeStruct((_N, _D // 2), jnp.int32),
    ],
)


def _dense1_body(agg1_ref, agg2_ref, w_ref, b_ref, alpha_ref, h1_ref, h2_ref,
                 hsum_ref):
    a = alpha_ref[0]
    w = w_ref[...]
    b = b_ref[...]
    y1 = jnp.dot(agg1_ref[0], w, preferred_element_type=jnp.float32) + b
    h1 = jnp.where(y1 >= 0, y1, a * y1)
    h1_ref[...] = h1
    y2 = jnp.dot(agg2_ref[0], w, preferred_element_type=jnp.float32) + b
    h2_ref[...] = jnp.where(y2 >= 0, y2, a * y2)

    @pl.when(pl.program_id(0) == 0)
    def _():
        hsum_ref[...] = jnp.zeros_like(hsum_ref)

    hsum_ref[...] += jnp.sum(h1, axis=0, keepdims=True)


_dense1 = pl.pallas_call(
    _dense1_body,
    grid=(_G,),
    in_specs=[
        pl.BlockSpec((1, _R, _D), lambda i: (0, i, 0)),
        pl.BlockSpec((1, _R, _D), lambda i: (1, i, 0)),
        pl.BlockSpec((_D, _H), lambda i: (0, 0)),
        pl.BlockSpec((1, _H), lambda i: (0, 0)),
        pl.BlockSpec(memory_space=pltpu.SMEM),
    ],
    out_specs=[
        pl.BlockSpec((_R, _H), lambda i: (i, 0)),
        pl.BlockSpec((_R, _H), lambda i: (i, 0)),
        pl.BlockSpec((1, _H), lambda i: (0, 0)),
    ],
    out_shape=[
        jax.ShapeDtypeStruct((_N, _H), jnp.float32),
        jax.ShapeDtypeStruct((_N, _H), jnp.float32),
        jax.ShapeDtypeStruct((1, _H), jnp.float32),
    ],
)


def _dense2_body(h1_ref, h2_ref, hsum_ref, wd_ref, cl_ref, sb1_ref, sb2_ref,
                 bd_ref, sc1_ref, sc2_ref, q_ref):
    cvec = jax.nn.sigmoid(hsum_ref[...] / _N)
    v = lax.dot_general(wd_ref[...], cvec, (((1,), (1,)), ((), ())),
                        preferred_element_type=jnp.float32)
    h1 = h1_ref[...]
    h2 = h2_ref[...]
    bd = bd_ref[0]
    sc1_ref[...] = (jnp.dot(h1, v, preferred_element_type=jnp.float32)
                    + bd + sb1_ref[...])
    sc2_ref[...] = (jnp.dot(h2, v, preferred_element_type=jnp.float32)
                    + bd + sb2_ref[...])
    cl = cl_ref[...]
    cross = lax.dot_general(h1, cl, (((1,), (1,)), ((), ())),
                            preferred_element_type=jnp.float32)
    h1s = jnp.sum(h1 * h1, axis=1, keepdims=True)
    cls = lax.dot_general(jnp.ones((1, _H), jnp.float32), cl * cl,
                          (((1,), (1,)), ((), ())),
                          preferred_element_type=jnp.float32)
    dist2 = h1s - 2.0 * cross + cls
    qn = 1.0 / (1.0 + dist2)
    q_ref[...] = qn / jnp.sum(qn, axis=1, keepdims=True)


_dense2 = pl.pallas_call(
    _dense2_body,
    grid=(_G,),
    in_specs=[
        pl.BlockSpec((_R, _H), lambda i: (i, 0)),
        pl.BlockSpec((_R, _H), lambda i: (i, 0)),
        pl.BlockSpec((1, _H), lambda i: (0, 0)),
        pl.BlockSpec((_H, _H), lambda i: (0, 0)),
        pl.BlockSpec((_K, _H), lambda i: (0, 0)),
        pl.BlockSpec((_R, 1), lambda i: (i, 0)),
        pl.BlockSpec((_R, 1), lambda i: (i, 0)),
        pl.BlockSpec(memory_space=pltpu.SMEM),
    ],
    out_specs=[
        pl.BlockSpec((_R, 1), lambda i: (i, 0)),
        pl.BlockSpec((_R, 1), lambda i: (i, 0)),
        pl.BlockSpec((_R, _K), lambda i: (i, 0)),
    ],
    out_shape=[
        jax.ShapeDtypeStruct((_N, 1), jnp.float32),
        jax.ShapeDtypeStruct((_N, 1), jnp.float32),
        jax.ShapeDtypeStruct((_N, _K), jnp.float32),
    ],
)


def kernel(seq1, seq2, adj_edge_index, adj_edge_weight, samp_bias1, samp_bias2,
           W_gcn, b_gcn, alpha, W_disc, b_disc, cluster_layer):
    t1, t2 = _prep(seq1, seq2)
    ei = adj_edge_index.astype(jnp.int32).reshape(2, _NS, _NG, _GC, _CH)
    w = adj_edge_weight.astype(jnp.float32).reshape(_NS, _NG, _GE)

    agg = _get_spmm()(t1, t2, ei, w)
    h1, h2, hsum = _dense1(agg, agg, W_gcn,
                           b_gcn.reshape(1, _H), alpha.reshape(1))
    sc1, sc2, q = _dense2(h1, h2, hsum, W_disc, cluster_layer,
                          samp_bias1.reshape(_N, 1), samp_bias2.reshape(_N, 1),
                          b_disc.reshape(1))
    ret = jnp.concatenate([sc1.reshape(1, _N), sc2.reshape(1, _N)], axis=1)
    return (ret, q, h1)


# speedup vs baseline: 1.2017x; 1.0252x over previous
import functools

import jax
import jax.numpy as jnp
from jax import lax
from jax.experimental import pallas as pl
from jax.experimental.pallas import tpu as pltpu
from jax.experimental.pallas import tpu_sc as plsc

_N = 10000
_E = 320000
_D = 128
_H = 512
_K = 20

_NC = 2
_NS = 16
_L = 16

_EPS = _E // _NS
_CH = 80
_GC = 25
_GE = _GC * _CH
_NG = _EPS // _GE
_SPS = 624
_WBC = ((0, 80), (80, 80), (160, 80), (240, 80), (320, 80), (400, 80),
        (480, 80), (560, 64))
_TAIL = _N - _NS * _SPS

@functools.cache
def _get_spmm():
    mesh = plsc.VectorSubcoreMesh(core_axis_name="c", subcore_axis_name="s",
                                  num_cores=_NC, num_subcores=_NS)
    return pl.kernel(
        _spmm_body,
        out_type=jax.ShapeDtypeStruct((_NC, _N, _D), jnp.float32),
        mesh=mesh,
        compiler_params=pltpu.CompilerParams(needs_layout_passes=False,
                                             use_tc_tiling_on_sc=False),
        scratch_types=[
            pltpu.VMEM_SHARED((_N, _D), jnp.float32),
            pltpu.VMEM((_GC, _CH), jnp.int32),
            pltpu.VMEM((_GC, _CH), jnp.int32),
            pltpu.VMEM((_GE,), jnp.float32),
            pltpu.VMEM((_CH, _D // 2), jnp.int32),
            pltpu.VMEM((_CH, _D // 2), jnp.int32),
            pltpu.VMEM((_CH, _D // 2), jnp.int32),
            pltpu.VMEM((_CH, _D), jnp.float32),
            pltpu.VMEM((_CH, _D), jnp.float32),
            pltpu.SemaphoreType.DMA,
            pltpu.SemaphoreType.DMA,
            pltpu.SemaphoreType.DMA,
            pltpu.SemaphoreType.DMA,
            pltpu.SemaphoreType.DMA,
        ],
    )


def _spmm_body(seq1h, seq2h, eih, wh, outh, acc, dstv, srcv, wv,
               gb0, gb1, gb2, st0, st1, gsem0, gsem1, gsem2, ssem0, ssem1):
    s = lax.axis_index("s")
    c = lax.axis_index("c")

    def zrow(j, carry):
        for k2 in range(_D // _L):
            st0[j, k2 * _L:(k2 + 1) * _L] = jnp.zeros((_L,), jnp.float32)
        return carry

    lax.fori_loop(0, _CH, zrow, 0)
    base = s * _SPS
    for off, nr in _WBC:
        pltpu.sync_copy(st0.at[pl.ds(0, nr)],
                        acc.at[pl.ds(base + off, nr)])

    @pl.when(s == 0)
    def _():
        pltpu.sync_copy(st0.at[pl.ds(0, _TAIL)],
                        acc.at[pl.ds(_NS * _SPS, _TAIL)])

    plsc.subcore_barrier()

    def core_prog(seqh, out2):
        gbufs = ((gb0, gsem0), (gb1, gsem1), (gb2, gsem2))
        sbufs = ((st0, ssem0), (st1, ssem1))

        def group(g, carry):
            pltpu.sync_copy(eih.at[0, s, g], dstv)
            pltpu.sync_copy(eih.at[1, s, g], srcv)
            pltpu.sync_copy(wh.at[s, g], wv)
            pltpu.async_copy(seqh.at[srcv.at[0]], gb0, gsem0)
            pltpu.async_copy(seqh.at[srcv.at[1]], gb1, gsem1)
            pltpu.async_copy(seqh.at[srcv.at[2]], gb2, gsem2)

            def chunk(t, c3):
                for b6 in range(6):
                    @pl.when(t % 6 == b6)
                    def _():
                        gb, gs = gbufs[b6 % 3]
                        sb, ss = sbufs[b6 % 2]
                        pltpu.make_async_copy(
                            seqh.at[srcv.at[t]], gb, gs).wait()

                        @pl.when(t >= 2)
                        def _():
                            pltpu.make_async_copy(
                                sb, acc.at[dstv.at[0]], ss).wait()

                        @plsc.parallel_loop(0, _CH, unroll=4)
                        def _(i):
                            wspl = plsc.load_gather(
                                wv, [jnp.broadcast_to(t * _CH + i, (_L,))])
                            for k2 in range(_D // 32):
                                x = gb[i, _L * k2:_L * (k2 + 1)]
                                fe = plsc.bitcast(x << 16, jnp.float32)
                                fo = plsc.bitcast(
                                    x & jnp.int32(-65536), jnp.float32)
                                sb[i, 32 * k2:32 * k2 + _L] = fe * wspl
                                sb[i, 32 * k2 + _L:32 * (k2 + 1)] = fo * wspl

                        pltpu.async_copy(sb, acc.at[dstv.at[t]], ss, add=True)

                        @pl.when(t + 3 < _GC)
                        def _():
                            pltpu.async_copy(
                                seqh.at[srcv.at[t + 3]], gb, gs)
                return c3

            lax.fori_loop(0, _GC, chunk, 0)
            for u in range(_GC - 2, _GC):
                sb, ss = sbufs[u % 2]
                pltpu.make_async_copy(sb, acc.at[dstv.at[0]], ss).wait()
            return carry

        lax.fori_loop(0, _NG, group, 0)
        plsc.subcore_barrier()
        for off, nr in _WBC:
            r0 = s * _SPS + off
            pltpu.sync_copy(acc.at[pl.ds(r0, nr)], st0.at[pl.ds(0, nr)])
            pltpu.sync_copy(st0.at[pl.ds(0, nr)], out2.at[pl.ds(r0, nr)])

        @pl.when(s == 0)
        def _():
            pltpu.sync_copy(acc.at[pl.ds(_NS * _SPS, _TAIL)],
                            st0.at[pl.ds(0, _TAIL)])
            pltpu.sync_copy(st0.at[pl.ds(0, _TAIL)],
                            out2.at[pl.ds(_NS * _SPS, _TAIL)])

    @pl.when(c == 0)
    def _():
        core_prog(seq1h, outh.at[0])

    @pl.when(c == 1)
    def _():
        core_prog(seq2h, outh.at[1])


_R = 1000
_G = _N // _R


def _rne_bf16_bits(v):
    u = lax.bitcast_convert_type(v, jnp.int32)
    r = u + jnp.int32(0x7FFF) + ((u >> 16) & jnp.int32(1))
    return r


def _prep_body(s1_ref, s2_ref, t1_ref, t2_ref):
    for x_ref, t_ref in ((s1_ref, t1_ref), (s2_ref, t2_ref)):
        x = x_ref[0]
        xl = jnp.concatenate(
            [x[:, 32 * k:32 * k + 16] for k in range(_D // 32)], axis=1)
        xh = jnp.concatenate(
            [x[:, 32 * k + 16:32 * k + 32] for k in range(_D // 32)], axis=1)
        lb = (_rne_bf16_bits(xl) >> 16) & jnp.int32(0xFFFF)
        hb = _rne_bf16_bits(xh) & jnp.int32(-65536)
        t_ref[...] = hb | lb


_prep = pl.pallas_call(
    _prep_body,
    grid=(_G,),
    in_specs=[
        pl.BlockSpec((1, _R, _D), lambda i: (0, i, 0)),
        pl.BlockSpec((1, _R, _D), lambda i: (0, i, 0)),
    ],
    out_specs=[
        pl.BlockSpec((_R, _D // 2), lambda i: (i, 0)),
        pl.BlockSpec((_R, _D // 2), lambda i: (i, 0)),
    ],
    out_shape=[
        jax.ShapeDtypeStruct((_N, _D // 2), jnp.int32),
        jax.ShapeDtypeStruct((_N, _D // 2), jnp.int32),
    ],
)


def _dense1_body(agg1_ref, w_ref, b_ref, alpha_ref, h1_ref, hsum_ref):
    a = alpha_ref[0]
    y1 = (jnp.dot(agg1_ref[0], w_ref[...], preferred_element_type=jnp.float32)
          + b_ref[...])
    h1 = jnp.where(y1 >= 0, y1, a * y1)
    h1_ref[...] = h1

    @pl.when(pl.program_id(0) == 0)
    def _():
        hsum_ref[...] = jnp.zeros_like(hsum_ref)

    hsum_ref[...] += jnp.sum(h1, axis=0, keepdims=True)


_dense1 = pl.pallas_call(
    _dense1_body,
    grid=(_G,),
    in_specs=[
        pl.BlockSpec((1, _R, _D), lambda i: (0, i, 0)),
        pl.BlockSpec((_D, _H), lambda i: (0, 0)),
        pl.BlockSpec((1, _H), lambda i: (0, 0)),
        pl.BlockSpec(memory_space=pltpu.SMEM),
    ],
    out_specs=[
        pl.BlockSpec((_R, _H), lambda i: (i, 0)),
        pl.BlockSpec((1, _H), lambda i: (0, 0)),
    ],
    out_shape=[
        jax.ShapeDtypeStruct((_N, _H), jnp.float32),
        jax.ShapeDtypeStruct((1, _H), jnp.float32),
    ],
)


def _dense2_body(agg1_ref, agg2_ref, w_ref, b_ref, alpha_ref, hsum_ref,
                 wd_ref, cl_ref, sb1_ref, sb2_ref, bd_ref,
                 sc1_ref, sc2_ref, q_ref):
    a = alpha_ref[0]
    w = w_ref[...]
    b = b_ref[...]
    y1 = jnp.dot(agg1_ref[0], w, preferred_element_type=jnp.float32) + b
    h1 = jnp.where(y1 >= 0, y1, a * y1)
    y2 = jnp.dot(agg2_ref[0], w, preferred_element_type=jnp.float32) + b
    h2 = jnp.where(y2 >= 0, y2, a * y2)
    cvec = jax.nn.sigmoid(hsum_ref[...] / _N)
    v = lax.dot_general(wd_ref[...], cvec, (((1,), (1,)), ((), ())),
                        preferred_element_type=jnp.float32)
    bd = bd_ref[0]
    sc1_ref[...] = (jnp.dot(h1, v, preferred_element_type=jnp.float32)
                    + bd + sb1_ref[...])
    sc2_ref[...] = (jnp.dot(h2, v, preferred_element_type=jnp.float32)
                    + bd + sb2_ref[...])
    cl = cl_ref[...]
    cross = lax.dot_general(h1, cl, (((1,), (1,)), ((), ())),
                            preferred_element_type=jnp.float32)
    h1s = jnp.sum(h1 * h1, axis=1, keepdims=True)
    cls = lax.dot_general(jnp.ones((1, _H), jnp.float32), cl * cl,
                          (((1,), (1,)), ((), ())),
                          preferred_element_type=jnp.float32)
    dist2 = h1s - 2.0 * cross + cls
    qn = 1.0 / (1.0 + dist2)
    q_ref[...] = qn / jnp.sum(qn, axis=1, keepdims=True)


_dense2 = pl.pallas_call(
    _dense2_body,
    grid=(_G,),
    in_specs=[
        pl.BlockSpec((1, _R, _D), lambda i: (0, i, 0)),
        pl.BlockSpec((1, _R, _D), lambda i: (1, i, 0)),
        pl.BlockSpec((_D, _H), lambda i: (0, 0)),
        pl.BlockSpec((1, _H), lambda i: (0, 0)),
        pl.BlockSpec(memory_space=pltpu.SMEM),
        pl.BlockSpec((1, _H), lambda i: (0, 0)),
        pl.BlockSpec((_H, _H), lambda i: (0, 0)),
        pl.BlockSpec((_K, _H), lambda i: (0, 0)),
        pl.BlockSpec((_R, 1), lambda i: (i, 0)),
        pl.BlockSpec((_R, 1), lambda i: (i, 0)),
        pl.BlockSpec(memory_space=pltpu.SMEM),
    ],
    out_specs=[
        pl.BlockSpec((_R, 1), lambda i: (i, 0)),
        pl.BlockSpec((_R, 1), lambda i: (i, 0)),
        pl.BlockSpec((_R, _K), lambda i: (i, 0)),
    ],
    out_shape=[
        jax.ShapeDtypeStruct((_N, 1), jnp.float32),
        jax.ShapeDtypeStruct((_N, 1), jnp.float32),
        jax.ShapeDtypeStruct((_N, _K), jnp.float32),
    ],
)


def kernel(seq1, seq2, adj_edge_index, adj_edge_weight, samp_bias1, samp_bias2,
           W_gcn, b_gcn, alpha, W_disc, b_disc, cluster_layer):
    t1, t2 = _prep(seq1, seq2)
    ei = adj_edge_index.astype(jnp.int32).reshape(2, _NS, _NG, _GC, _CH)
    w = adj_edge_weight.astype(jnp.float32).reshape(_NS, _NG, _GE)

    agg = _get_spmm()(t1, t2, ei, w)
    h1, hsum = _dense1(agg, W_gcn, b_gcn.reshape(1, _H), alpha.reshape(1))
    sc1, sc2, q = _dense2(agg, agg, W_gcn, b_gcn.reshape(1, _H),
                          alpha.reshape(1), hsum, W_disc, cluster_layer,
                          samp_bias1.reshape(_N, 1), samp_bias2.reshape(_N, 1),
                          b_disc.reshape(1))
    ret = jnp.concatenate([sc1.reshape(1, _N), sc2.reshape(1, _N)], axis=1)
    return (ret, q, h1)


# speedup vs baseline: 1.2058x; 1.0034x over previous
import functools

import jax
import jax.numpy as jnp
from jax import lax
from jax.experimental import pallas as pl
from jax.experimental.pallas import tpu as pltpu
from jax.experimental.pallas import tpu_sc as plsc

_N = 10000
_E = 320000
_D = 128
_H = 512
_K = 20

_NC = 2
_NS = 16
_L = 16

_EPS = _E // _NS
_CH = 80
_GC = 25
_GE = _GC * _CH
_NG = _EPS // _GE
_SPS = 624
_WBC = ((0, 80), (80, 80), (160, 80), (240, 80), (320, 80), (400, 80),
        (480, 80), (560, 64))
_TAIL = _N - _NS * _SPS

@functools.cache
def _get_spmm():
    mesh = plsc.VectorSubcoreMesh(core_axis_name="c", subcore_axis_name="s",
                                  num_cores=_NC, num_subcores=_NS)
    return pl.kernel(
        _spmm_body,
        out_type=jax.ShapeDtypeStruct((_NC, _N, _D), jnp.float32),
        mesh=mesh,
        compiler_params=pltpu.CompilerParams(needs_layout_passes=False,
                                             use_tc_tiling_on_sc=False),
        scratch_types=[
            pltpu.VMEM_SHARED((_N, _D), jnp.float32),
            pltpu.VMEM((_GC, _CH), jnp.int32),
            pltpu.VMEM((_GC, _CH), jnp.int32),
            pltpu.VMEM((_GE,), jnp.float32),
            pltpu.VMEM((_CH, _D // 2), jnp.int32),
            pltpu.VMEM((_CH, _D // 2), jnp.int32),
            pltpu.VMEM((_CH, _D // 2), jnp.int32),
            pltpu.VMEM((_CH, _D), jnp.float32),
            pltpu.VMEM((_CH, _D), jnp.float32),
            pltpu.SemaphoreType.DMA,
            pltpu.SemaphoreType.DMA,
            pltpu.SemaphoreType.DMA,
            pltpu.SemaphoreType.DMA,
            pltpu.SemaphoreType.DMA,
        ],
    )


def _spmm_body(seq1h, seq2h, eih, wh, outh, acc, dstv, srcv, wv,
               gb0, gb1, gb2, st0, st1, gsem0, gsem1, gsem2, ssem0, ssem1):
    s = lax.axis_index("s")
    c = lax.axis_index("c")

    def zrow(j, carry):
        for k2 in range(_D // _L):
            st0[j, k2 * _L:(k2 + 1) * _L] = jnp.zeros((_L,), jnp.float32)
        return carry

    lax.fori_loop(0, _CH, zrow, 0)
    base = s * _SPS
    for off, nr in _WBC:
        pltpu.sync_copy(st0.at[pl.ds(0, nr)],
                        acc.at[pl.ds(base + off, nr)])

    @pl.when(s == 0)
    def _():
        pltpu.sync_copy(st0.at[pl.ds(0, _TAIL)],
                        acc.at[pl.ds(_NS * _SPS, _TAIL)])

    plsc.subcore_barrier()

    def core_prog(seqh, out2):
        gbufs = ((gb0, gsem0), (gb1, gsem1), (gb2, gsem2))
        sbufs = ((st0, ssem0), (st1, ssem1))

        def group(g, carry):
            pltpu.sync_copy(eih.at[0, s, g], dstv)
            pltpu.sync_copy(eih.at[1, s, g], srcv)
            pltpu.sync_copy(wh.at[s, g], wv)
            pltpu.async_copy(seqh.at[srcv.at[0]], gb0, gsem0)
            pltpu.async_copy(seqh.at[srcv.at[1]], gb1, gsem1)
            pltpu.async_copy(seqh.at[srcv.at[2]], gb2, gsem2)

            def chunk(t, c3):
                for b6 in range(6):
                    @pl.when(t % 6 == b6)
                    def _():
                        gb, gs = gbufs[b6 % 3]
                        sb, ss = sbufs[b6 % 2]
                        pltpu.make_async_copy(
                            seqh.at[srcv.at[t]], gb, gs).wait()

                        @pl.when(t >= 2)
                        def _():
                            pltpu.make_async_copy(
                                sb, acc.at[dstv.at[0]], ss).wait()

                        @plsc.parallel_loop(0, _CH, unroll=4)
                        def _(i):
                            wspl = plsc.load_gather(
                                wv, [jnp.broadcast_to(t * _CH + i, (_L,))])
                            for k2 in range(_D // 32):
                                x = gb[i, _L * k2:_L * (k2 + 1)]
                                fe = plsc.bitcast(x << 16, jnp.float32)
                                fo = plsc.bitcast(
                                    x & jnp.int32(-65536), jnp.float32)
                                sb[i, 32 * k2:32 * k2 + _L] = fe * wspl
                                sb[i, 32 * k2 + _L:32 * (k2 + 1)] = fo * wspl

                        pltpu.async_copy(sb, acc.at[dstv.at[t]], ss, add=True)

                        @pl.when(t + 3 < _GC)
                        def _():
                            pltpu.async_copy(
                                seqh.at[srcv.at[t + 3]], gb, gs)
                return c3

            lax.fori_loop(0, _GC, chunk, 0)
            for u in range(_GC - 2, _GC):
                sb, ss = sbufs[u % 2]
                pltpu.make_async_copy(sb, acc.at[dstv.at[0]], ss).wait()
            return carry

        lax.fori_loop(0, _NG, group, 0)
        plsc.subcore_barrier()
        for off, nr in _WBC:
            r0 = s * _SPS + off
            pltpu.sync_copy(acc.at[pl.ds(r0, nr)], st0.at[pl.ds(0, nr)])
            pltpu.sync_copy(st0.at[pl.ds(0, nr)], out2.at[pl.ds(r0, nr)])

        @pl.when(s == 0)
        def _():
            pltpu.sync_copy(acc.at[pl.ds(_NS * _SPS, _TAIL)],
                            st0.at[pl.ds(0, _TAIL)])
            pltpu.sync_copy(st0.at[pl.ds(0, _TAIL)],
                            out2.at[pl.ds(_NS * _SPS, _TAIL)])

    @pl.when(c == 0)
    def _():
        core_prog(seq1h, outh.at[0])

    @pl.when(c == 1)
    def _():
        core_prog(seq2h, outh.at[1])


_R = 1000
_G = _N // _R


def _rne_bf16_bits(v):
    u = lax.bitcast_convert_type(v, jnp.int32)
    r = u + jnp.int32(0x7FFF) + ((u >> 16) & jnp.int32(1))
    return r


def _prep_body(s1_ref, s2_ref, t1_ref, t2_ref):
    for x_ref, t_ref in ((s1_ref, t1_ref), (s2_ref, t2_ref)):
        x = x_ref[0]
        for k in range(_D // 32):
            xl = x[:, 32 * k:32 * k + 16]
            xh = x[:, 32 * k + 16:32 * k + 32]
            lb = (_rne_bf16_bits(xl) >> 16) & jnp.int32(0xFFFF)
            hb = _rne_bf16_bits(xh) & jnp.int32(-65536)
            t_ref[:, 16 * k:16 * (k + 1)] = hb | lb


_prep = pl.pallas_call(
    _prep_body,
    grid=(_G,),
    in_specs=[
        pl.BlockSpec((1, _R, _D), lambda i: (0, i, 0)),
        pl.BlockSpec((1, _R, _D), lambda i: (0, i, 0)),
    ],
    out_specs=[
        pl.BlockSpec((_R, _D // 2), lambda i: (i, 0)),
        pl.BlockSpec((_R, _D // 2), lambda i: (i, 0)),
    ],
    out_shape=[
        jax.ShapeDtypeStruct((_N, _D // 2), jnp.int32),
        jax.ShapeDtypeStruct((_N, _D // 2), jnp.int32),
    ],
)


def _dense1_body(agg1_ref, w_ref, b_ref, alpha_ref, wd_ref, cl_ref,
                 h1_ref, hsum_ref, v_ref, cls_ref):
    a = alpha_ref[0]
    y1 = (jnp.dot(agg1_ref[0], w_ref[...], preferred_element_type=jnp.float32)
          + b_ref[...])
    h1 = jnp.where(y1 >= 0, y1, a * y1)
    h1_ref[...] = h1

    @pl.when(pl.program_id(0) == 0)
    def _():
        hsum_ref[...] = jnp.zeros_like(hsum_ref)

    hsum_ref[...] += jnp.sum(h1, axis=0, keepdims=True)

    @pl.when(pl.program_id(0) == _G - 1)
    def _():
        cvec = jax.nn.sigmoid(hsum_ref[...] / _N)
        v_ref[...] = lax.dot_general(
            wd_ref[...], cvec, (((1,), (1,)), ((), ())),
            preferred_element_type=jnp.float32)
        cl = cl_ref[...]
        cls_ref[...] = lax.dot_general(
            jnp.ones((1, _H), jnp.float32), cl * cl, (((1,), (1,)), ((), ())),
            preferred_element_type=jnp.float32)


_dense1 = pl.pallas_call(
    _dense1_body,
    grid=(_G,),
    in_specs=[
        pl.BlockSpec((1, _R, _D), lambda i: (0, i, 0)),
        pl.BlockSpec((_D, _H), lambda i: (0, 0)),
        pl.BlockSpec((1, _H), lambda i: (0, 0)),
        pl.BlockSpec(memory_space=pltpu.SMEM),
        pl.BlockSpec((_H, _H), lambda i: (0, 0)),
        pl.BlockSpec((_K, _H), lambda i: (0, 0)),
    ],
    out_specs=[
        pl.BlockSpec((_R, _H), lambda i: (i, 0)),
        pl.BlockSpec((1, _H), lambda i: (0, 0)),
        pl.BlockSpec((_H, 1), lambda i: (0, 0)),
        pl.BlockSpec((1, _K), lambda i: (0, 0)),
    ],
    out_shape=[
        jax.ShapeDtypeStruct((_N, _H), jnp.float32),
        jax.ShapeDtypeStruct((1, _H), jnp.float32),
        jax.ShapeDtypeStruct((_H, 1), jnp.float32),
        jax.ShapeDtypeStruct((1, _K), jnp.float32),
    ],
)


def _dense2_body(agg1_ref, agg2_ref, w_ref, b_ref, alpha_ref, v_ref,
                 cls_ref, cl_ref, sb1_ref, sb2_ref, bd_ref,
                 sc1_ref, sc2_ref, q_ref):
    a = alpha_ref[0]
    w = w_ref[...]
    b = b_ref[...]
    y1 = jnp.dot(agg1_ref[0], w, preferred_element_type=jnp.float32) + b
    h1 = jnp.where(y1 >= 0, y1, a * y1)
    y2 = jnp.dot(agg2_ref[0], w, preferred_element_type=jnp.float32) + b
    h2 = jnp.where(y2 >= 0, y2, a * y2)
    v = v_ref[...]
    bd = bd_ref[0]
    sc1_ref[...] = (jnp.dot(h1, v, preferred_element_type=jnp.float32)
                    + bd + sb1_ref[...])
    sc2_ref[...] = (jnp.dot(h2, v, preferred_element_type=jnp.float32)
                    + bd + sb2_ref[...])
    cl = cl_ref[...]
    cross = lax.dot_general(h1, cl, (((1,), (1,)), ((), ())),
                            preferred_element_type=jnp.float32)
    h1s = jnp.sum(h1 * h1, axis=1, keepdims=True)
    dist2 = h1s - 2.0 * cross + cls_ref[...]
    qn = 1.0 / (1.0 + dist2)
    q_ref[...] = qn / jnp.sum(qn, axis=1, keepdims=True)


_dense2 = pl.pallas_call(
    _dense2_body,
    grid=(_G,),
    in_specs=[
        pl.BlockSpec((1, _R, _D), lambda i: (0, i, 0)),
        pl.BlockSpec((1, _R, _D), lambda i: (1, i, 0)),
        pl.BlockSpec((_D, _H), lambda i: (0, 0)),
        pl.BlockSpec((1, _H), lambda i: (0, 0)),
        pl.BlockSpec(memory_space=pltpu.SMEM),
        pl.BlockSpec((_H, 1), lambda i: (0, 0)),
        pl.BlockSpec((1, _K), lambda i: (0, 0)),
        pl.BlockSpec((_K, _H), lambda i: (0, 0)),
        pl.BlockSpec((_R, 1), lambda i: (i, 0)),
        pl.BlockSpec((_R, 1), lambda i: (i, 0)),
        pl.BlockSpec(memory_space=pltpu.SMEM),
    ],
    out_specs=[
        pl.BlockSpec((_R, 1), lambda i: (i, 0)),
        pl.BlockSpec((_R, 1), lambda i: (i, 0)),
        pl.BlockSpec((_R, _K), lambda i: (i, 0)),
    ],
    out_shape=[
        jax.ShapeDtypeStruct((_N, 1), jnp.float32),
        jax.ShapeDtypeStruct((_N, 1), jnp.float32),
        jax.ShapeDtypeStruct((_N, _K), jnp.float32),
    ],
)


def kernel(seq1, seq2, adj_edge_index, adj_edge_weight, samp_bias1, samp_bias2,
           W_gcn, b_gcn, alpha, W_disc, b_disc, cluster_layer):
    t1, t2 = _prep(seq1, seq2)
    ei = adj_edge_index.astype(jnp.int32).reshape(2, _NS, _NG, _GC, _CH)
    w = adj_edge_weight.astype(jnp.float32).reshape(_NS, _NG, _GE)

    agg = _get_spmm()(t1, t2, ei, w)
    h1, hsum, v, cls = _dense1(agg, W_gcn, b_gcn.reshape(1, _H),
                               alpha.reshape(1), W_disc, cluster_layer)
    sc1, sc2, q = _dense2(agg, agg, W_gcn, b_gcn.reshape(1, _H),
                          alpha.reshape(1), v, cls, cluster_layer,
                          samp_bias1.reshape(_N, 1), samp_bias2.reshape(_N, 1),
                          b_disc.reshape(1))
    ret = jnp.concatenate([sc1.reshape(1, _N), sc2.reshape(1, _N)], axis=1)
    return (ret, q, h1)
